# baseline (speedup 1.0000x reference)
import functools

import jax
import jax.numpy as jnp
from jax import lax
from jax.experimental import pallas as pl
from jax.experimental.pallas import tpu as pltpu
from jax.experimental.pallas import tpu_sc as plsc

_GROUP = 2736
_NGROUP = 3
_HIST_LO = 128

_SC_CORES = 2
_SC_SUBCORES = 16
_SC_WORKERS = _SC_CORES * _SC_SUBCORES
_SC_CHUNK = 128


def _dist_argmin_body(n_codes, zsq_ref, z_ref, cb_ref, idx_ref, perp_ref,
                      runmin_ref, runidx_ref, hist_ref, d_ref):
    b = pl.program_id(0)
    k = pl.program_id(1)
    nb = pl.num_programs(0)
    nk = pl.num_programs(1)
    zb = z_ref[0]
    cbk = cb_ref[...]
    kb, hw = cbk.shape[0], zb.shape[1]

    mm = lax.dot_general(cbk.astype(jnp.bfloat16), zb.astype(jnp.bfloat16),
                         (((1,), (0,)), ((), ())),
                         preferred_element_type=jnp.float32)
    zsq = zsq_ref[0]
    csq = jnp.sum(cbk * cbk, axis=1, keepdims=True)
    d = (zsq + csq) - 2.0 * mm

    sub_iota = lax.broadcasted_iota(jnp.int32, (8, hw), 0)
    d_ref[...] = d

    def _fold(r, carry):
        minv, idxv = carry
        dr = d_ref[pl.ds(r * 8, 8), :]
        take = dr < minv
        return (jnp.where(take, dr, minv),
                jnp.where(take, sub_iota + r * 8, idxv))

    minv0 = jnp.full((8, hw), jnp.inf, jnp.float32)
    idxv0 = jnp.zeros((8, hw), jnp.int32)
    minv, idxv = lax.fori_loop(0, kb // 8, _fold, (minv0, idxv0))
    bmin = jnp.min(minv, axis=0, keepdims=True)
    barg = jnp.min(jnp.where(minv == bmin, idxv, kb * nk),
                   axis=0, keepdims=True) + k * kb

    @pl.when(k == 0)
    def _():
        runmin_ref[...] = bmin.astype(jnp.bfloat16).astype(jnp.float32)
        runidx_ref[...] = barg

    @pl.when(k > 0)
    def _():
        upd = bmin < runmin_ref[...]
        runidx_ref[...] = jnp.where(upd, barg, runidx_ref[...])
        sel = jnp.where(upd, bmin, runmin_ref[...])
        runmin_ref[...] = sel.astype(jnp.bfloat16).astype(jnp.float32)

    @pl.when(k == nk - 1)
    def _():
        fi = runidx_ref[...]
        idx_ref[...] = fi.reshape(1, 1, hw)
        hist_hi = hist_ref.shape[0]
        hi = lax.shift_right_arithmetic(fi, 7)
        lo = jnp.bitwise_and(fi, _HIST_LO - 1)
        oh_hi = (lax.broadcasted_iota(jnp.int32, (hist_hi, hw), 0)
                 == hi).astype(jnp.float32)
        oh_lo = (lax.broadcasted_iota(jnp.int32, (_HIST_LO, hw), 0)
                 == lo).astype(jnp.float32)
        h2 = lax.dot_general(oh_hi, oh_lo, (((1,), (1,)), ((), ())),
                             preferred_element_type=jnp.float32)

        @pl.when(b == 0)
        def _():
            hist_ref[...] = h2

        @pl.when(b > 0)
        def _():
            hist_ref[...] = hist_ref[...] + h2

        @pl.when(b == nb - 1)
        def _():
            p = hist_ref[...] / jnp.float32(nb * hw)
            plogp = p * jnp.log(jnp.clip(p, 1e-10, None))
            ent = jnp.sum(plogp, axis=(0, 1), keepdims=True)
            perp_ref[...] = jnp.exp(-ent)


def _tc_dist_argmin(zsq3, z3, cb_pad, n_codes):
    B, D, HW = z3.shape
    grid = (B, _NGROUP)
    return pl.pallas_call(
        functools.partial(_dist_argmin_body, n_codes),
        grid=grid,
        in_specs=[
            pl.BlockSpec((1, 1, HW), lambda b, k: (b, 0, 0)),
            pl.BlockSpec((1, D, HW), lambda b, k: (b, 0, 0)),
            pl.BlockSpec((_GROUP, D), lambda b, k: (k, 0)),
        ],
        out_specs=[
            pl.BlockSpec((1, 1, HW), lambda b, k: (b, 0, 0)),
            pl.BlockSpec((1, 1), lambda b, k: (0, 0)),
        ],
        out_shape=[
            jax.ShapeDtypeStruct((B, 1, HW), jnp.int32),
            jax.ShapeDtypeStruct((1, 1), jnp.float32),
        ],
        scratch_shapes=[
            pltpu.VMEM((1, HW), jnp.float32),
            pltpu.VMEM((1, HW), jnp.int32),
            pltpu.VMEM((n_codes // _HIST_LO, _HIST_LO), jnp.float32),
            pltpu.VMEM((_GROUP, HW), jnp.float32),
        ],
    )(zsq3, z3, cb_pad)


def _sc_gather(codebook, indices):
    n_rows = indices.shape[0]
    d = codebook.shape[1]
    rows_per_worker = n_rows // _SC_WORKERS
    chunks = rows_per_worker // _SC_CHUNK
    mesh = plsc.VectorSubcoreMesh(core_axis_name="c", subcore_axis_name="s")

    @functools.partial(
        pl.kernel,
        mesh=mesh,
        out_type=jax.ShapeDtypeStruct((n_rows, d), jnp.float32),
        scratch_types=[
            pltpu.VMEM((_SC_CHUNK,), jnp.int32),
            pltpu.VMEM((_SC_CHUNK, d), jnp.float32),
            pltpu.SemaphoreType.DMA,
        ],
    )
    def gather_kernel(table_hbm, idx_hbm, out_hbm, idx_v, rows_v, sem):
        wid = lax.axis_index("s") * _SC_CORES + lax.axis_index("c")
        base0 = wid * rows_per_worker

        @pl.loop(0, chunks)
        def _(c):
            base = base0 + c * _SC_CHUNK
            pltpu.sync_copy(idx_hbm.at[pl.ds(base, _SC_CHUNK)], idx_v)
            pltpu.async_copy(table_hbm.at[idx_v], rows_v, sem).wait()
            pltpu.sync_copy(rows_v, out_hbm.at[pl.ds(base, _SC_CHUNK)])

    return gather_kernel(codebook, indices)


def kernel(z, codebook):
    B, C, H, W = z.shape
    K, D = codebook.shape
    HW = H * W
    flat_z = jnp.transpose(z, (0, 2, 3, 1)).reshape(-1, C)
    zsq3 = jnp.sum(flat_z**2, axis=1).reshape(B, 1, HW)
    pad = jnp.zeros((_NGROUP * _GROUP - K, D), jnp.float32)
    pad = pad.at[:, 0].set(128.0)
    cb_pad = jnp.concatenate([codebook, pad], axis=0)
    z3 = z.reshape(B, C, HW)
    idx3, perp = _tc_dist_argmin(zsq3, z3, cb_pad, K)
    indices = idx3.reshape(B * HW)
    rows = _sc_gather(codebook, indices)
    q = jnp.transpose(rows.reshape(B, H, W, C), (0, 3, 1, 2))
    q_st = z + lax.stop_gradient(q - z)
    return (q, q_st, indices, perp.reshape(()))


# speedup vs baseline: 1.2311x; 1.2311x over previous
import functools

import jax
import jax.numpy as jnp
from jax import lax
from jax.experimental import pallas as pl
from jax.experimental.pallas import tpu as pltpu
from jax.experimental.pallas import tpu_sc as plsc

_GROUP = 2736
_NGROUP = 3
_HIST_LO = 128

_SC_CORES = 2
_SC_SUBCORES = 16
_SC_WORKERS = _SC_CORES * _SC_SUBCORES
_SC_CHUNK = 128


def _dist_argmin_body(n_codes, zsq_ref, z_ref, cb_ref, idx_ref, perp_ref,
                      runmin_ref, runidx_ref, hist_ref):
    b = pl.program_id(0)
    k = pl.program_id(1)
    nb = pl.num_programs(0)
    nk = pl.num_programs(1)
    zb = z_ref[0]
    cbk = cb_ref[...]
    kb, hw = cbk.shape[0], zb.shape[1]

    mm = lax.dot_general(cbk.astype(jnp.bfloat16), zb.astype(jnp.bfloat16),
                         (((1,), (0,)), ((), ())),
                         preferred_element_type=jnp.float32)
    zsq = zsq_ref[0]
    csq = jnp.sum(cbk * cbk, axis=1, keepdims=True)
    d = (zsq + csq) - 2.0 * mm

    bmin = jnp.min(d, axis=0, keepdims=True)
    row = lax.broadcasted_iota(jnp.int32, d.shape, 0)
    barg = jnp.min(jnp.where(d == bmin, row, kb * nk),
                   axis=0, keepdims=True) + k * kb

    @pl.when(k == 0)
    def _():
        runmin_ref[...] = bmin.astype(jnp.bfloat16).astype(jnp.float32)
        runidx_ref[...] = barg

    @pl.when(k > 0)
    def _():
        upd = bmin < runmin_ref[...]
        runidx_ref[...] = jnp.where(upd, barg, runidx_ref[...])
        sel = jnp.where(upd, bmin, runmin_ref[...])
        runmin_ref[...] = sel.astype(jnp.bfloat16).astype(jnp.float32)

    @pl.when(k == nk - 1)
    def _():
        fi = runidx_ref[...]
        idx_ref[...] = fi.reshape(1, 1, hw)
        hist_hi = hist_ref.shape[0]
        hi = lax.shift_right_arithmetic(fi, 7)
        lo = jnp.bitwise_and(fi, _HIST_LO - 1)
        oh_hi = (lax.broadcasted_iota(jnp.int32, (hist_hi, hw), 0)
                 == hi).astype(jnp.float32)
        oh_lo = (lax.broadcasted_iota(jnp.int32, (_HIST_LO, hw), 0)
                 == lo).astype(jnp.float32)
        h2 = lax.dot_general(oh_hi, oh_lo, (((1,), (1,)), ((), ())),
                             preferred_element_type=jnp.float32)

        @pl.when(b == 0)
        def _():
            hist_ref[...] = h2

        @pl.when(b > 0)
        def _():
            hist_ref[...] = hist_ref[...] + h2

        @pl.when(b == nb - 1)
        def _():
            p = hist_ref[...] / jnp.float32(nb * hw)
            plogp = p * jnp.log(jnp.clip(p, 1e-10, None))
            ent = jnp.sum(plogp, axis=(0, 1), keepdims=True)
            perp_ref[...] = jnp.exp(-ent)


def _tc_dist_argmin(zsq3, z3, cb_pad, n_codes):
    B, D, HW = z3.shape
    grid = (B, _NGROUP)
    return pl.pallas_call(
        functools.partial(_dist_argmin_body, n_codes),
        grid=grid,
        in_specs=[
            pl.BlockSpec((1, 1, HW), lambda b, k: (b, 0, 0)),
            pl.BlockSpec((1, D, HW), lambda b, k: (b, 0, 0)),
            pl.BlockSpec((_GROUP, D), lambda b, k: (k, 0)),
        ],
        out_specs=[
            pl.BlockSpec((1, 1, HW), lambda b, k: (b, 0, 0)),
            pl.BlockSpec((1, 1), lambda b, k: (0, 0)),
        ],
        out_shape=[
            jax.ShapeDtypeStruct((B, 1, HW), jnp.int32),
            jax.ShapeDtypeStruct((1, 1), jnp.float32),
        ],
        scratch_shapes=[
            pltpu.VMEM((1, HW), jnp.float32),
            pltpu.VMEM((1, HW), jnp.int32),
            pltpu.VMEM((n_codes // _HIST_LO, _HIST_LO), jnp.float32),
        ],
    )(zsq3, z3, cb_pad)


def _sc_gather(codebook, indices):
    n_rows = indices.shape[0]
    d = codebook.shape[1]
    rows_per_worker = n_rows // _SC_WORKERS
    chunks = rows_per_worker // _SC_CHUNK
    mesh = plsc.VectorSubcoreMesh(core_axis_name="c", subcore_axis_name="s")

    @functools.partial(
        pl.kernel,
        mesh=mesh,
        out_type=jax.ShapeDtypeStruct((n_rows, d), jnp.float32),
        scratch_types=[
            pltpu.VMEM((_SC_CHUNK,), jnp.int32),
            pltpu.VMEM((_SC_CHUNK, d), jnp.float32),
            pltpu.SemaphoreType.DMA,
        ],
    )
    def gather_kernel(table_hbm, idx_hbm, out_hbm, idx_v, rows_v, sem):
        wid = lax.axis_index("s") * _SC_CORES + lax.axis_index("c")
        base0 = wid * rows_per_worker

        @pl.loop(0, chunks)
        def _(c):
            base = base0 + c * _SC_CHUNK
            pltpu.sync_copy(idx_hbm.at[pl.ds(base, _SC_CHUNK)], idx_v)
            pltpu.async_copy(table_hbm.at[idx_v], rows_v, sem).wait()
            pltpu.sync_copy(rows_v, out_hbm.at[pl.ds(base, _SC_CHUNK)])

    return gather_kernel(codebook, indices)


def kernel(z, codebook):
    B, C, H, W = z.shape
    K, D = codebook.shape
    HW = H * W
    flat_z = jnp.transpose(z, (0, 2, 3, 1)).reshape(-1, C)
    zsq3 = jnp.sum(flat_z**2, axis=1).reshape(B, 1, HW)
    pad = jnp.zeros((_NGROUP * _GROUP - K, D), jnp.float32)
    pad = pad.at[:, 0].set(128.0)
    cb_pad = jnp.concatenate([codebook, pad], axis=0)
    z3 = z.reshape(B, C, HW)
    idx3, perp = _tc_dist_argmin(zsq3, z3, cb_pad, K)
    indices = idx3.reshape(B * HW)
    rows = _sc_gather(codebook, indices)
    q = jnp.transpose(rows.reshape(B, H, W, C), (0, 3, 1, 2))
    q_st = z + lax.stop_gradient(q - z)
    return (q, q_st, indices, perp.reshape(()))
